# baseline (speedup 1.0000x reference)
import functools

import jax
import jax.numpy as jnp
from jax.experimental import pallas as pl
from jax.experimental.pallas import tpu as pltpu

K = 50
SUB = 8


def _topk_kernel(p_blk_ref, pt_ref, idx_ref, *, n):
    p_blk = p_blk_ref[...]
    pt = pt_ref[...]
    r = p_blk.shape[0]

    sq_r = jnp.sum(p_blk * p_blk, axis=1, keepdims=True)
    sq_c = jnp.sum(pt * pt, axis=0, keepdims=True)
    dot = jax.lax.dot_general(
        p_blk, pt, (((1,), (0,)), ((), ())),
        preferred_element_type=jnp.float32,
    )
    d2 = sq_r + sq_c - 2.0 * dot

    lane = jax.lax.broadcasted_iota(jnp.int32, (r, n), 1)

    def body(j, d):
        am = jnp.argmin(d, axis=1).astype(jnp.int32)
        idx_ref[pl.ds(j, 1), :] = am[None, :]
        return jnp.where(lane == am[:, None], jnp.inf, d)

    jax.lax.fori_loop(0, K, body, d2)


def _knn_topk(p):
    n = p.shape[0]
    blk = 256
    grid = n // blk
    pt = p.T

    idx_km = pl.pallas_call(
        functools.partial(_topk_kernel, n=n),
        grid=(grid,),
        in_specs=[
            pl.BlockSpec((blk, 3), lambda i: (i, 0)),
            pl.BlockSpec((3, n), lambda i: (0, 0)),
        ],
        out_specs=pl.BlockSpec((K, blk), lambda i: (0, i)),
        out_shape=jax.ShapeDtypeStruct((K, n), jnp.int32),
    )(p, pt)
    return idx_km.T


def _rot3(A, V, p, q):
    app, aqq, apq = A[p][p], A[q][q], A[p][q]
    rho = (aqq - app) / (2.0 * apq)
    sgn = jnp.where(rho >= 0.0, 1.0, -1.0)
    t = sgn / (jnp.abs(rho) + jnp.sqrt(1.0 + rho * rho))
    t = jnp.where(apq == 0.0, 0.0, t)
    c = 1.0 / jnp.sqrt(1.0 + t * t)
    s = t * c
    for k in range(3):
        ap, aq = A[p][k], A[q][k]
        A[p][k] = c * ap - s * aq
        A[q][k] = s * ap + c * aq
    for k in range(3):
        ap, aq = A[k][p], A[k][q]
        A[k][p] = c * ap - s * aq
        A[k][q] = s * ap + c * aq
    for k in range(3):
        vp, vq = V[k][p], V[k][q]
        V[k][p] = c * vp - s * vq
        V[k][q] = s * vp + c * vq


def _frames_kernel(cov_ref, out_ref):
    xx, xy, xz, yy, yz, zz = (cov_ref[i] for i in range(6))

    A = [[None] * 3 for _ in range(3)]
    A[0][0] = xx
    A[1][1] = yy
    A[2][2] = zz
    A[0][1] = A[1][0] = xy
    A[0][2] = A[2][0] = xz
    A[1][2] = A[2][1] = yz

    one = jnp.ones_like(A[0][0])
    zero = jnp.zeros_like(A[0][0])
    V = [[one if i == j else zero for j in range(3)] for i in range(3)]

    for _ in range(9):
        _rot3(A, V, 0, 2)
        _rot3(A, V, 1, 2)
        _rot3(A, V, 0, 1)

    w = [A[0][0], A[1][1], A[2][2]]
    cols = [[V[r][c] for r in range(3)] for c in range(3)]

    def cswap(i, j):
        do = w[i] > w[j]
        w[i], w[j] = (jnp.where(do, w[j], w[i]), jnp.where(do, w[i], w[j]))
        for r in range(3):
            a, b = cols[i][r], cols[j][r]
            cols[i][r] = jnp.where(do, b, a)
            cols[j][r] = jnp.where(do, a, b)

    cswap(0, 1)
    cswap(1, 2)
    cswap(0, 1)

    nrm, t1, t2 = cols[0], cols[1], cols[2]
    det = (
        nrm[0] * (t1[1] * t2[2] - t1[2] * t2[1])
        - nrm[1] * (t1[0] * t2[2] - t1[2] * t2[0])
        + nrm[2] * (t1[0] * t2[1] - t1[1] * t2[0])
    )
    for r in range(3):
        out_ref[r] = nrm[r]
        out_ref[3 + r] = t1[r] * det
        out_ref[6 + r] = t2[r]


def _curv_kernel(s2_ref, m_ref, fr_ref, out_ref):
    xx, xy, xz, yy, yz, zz = (s2_ref[i] for i in range(6))
    S2 = [[xx, xy, xz], [xy, yy, yz], [xz, yz, zz]]
    M = [[m_ref[3 * i + j] for j in range(3)] for i in range(3)]
    t1 = [fr_ref[3 + r] for r in range(3)]
    t2 = [fr_ref[6 + r] for r in range(3)]

    def quad(u, B, v):
        acc = None
        for i in range(3):
            row = None
            for j in range(3):
                term = B[i][j] * v[j]
                row = term if row is None else row + term
            term = u[i] * row
            acc = term if acc is None else acc + term
        return acc

    Sm = [[M[i][j] + M[j][i] for j in range(3)] for i in range(3)]

    x11 = quad(t1, S2, t1)
    x12 = quad(t1, S2, t2)
    x22 = quad(t2, S2, t2)
    s11 = quad(t1, Sm, t1)
    s12 = quad(t1, Sm, t2)
    s22 = quad(t2, Sm, t2)

    rho = (x22 - x11) / (2.0 * x12)
    sgn = jnp.where(rho >= 0.0, 1.0, -1.0)
    t = sgn / (jnp.abs(rho) + jnp.sqrt(1.0 + rho * rho))
    t = jnp.where(x12 == 0.0, 0.0, t)
    c = 1.0 / jnp.sqrt(1.0 + t * t)
    s = t * c
    w0 = c * (c * x11 - s * x12) - s * (c * x12 - s * x22)
    w1 = s * (s * x11 + c * x12) + c * (s * x12 + c * x22)
    q00, q01, q10, q11 = c, s, -s, c
    do = w0 > w1
    a = jnp.where(do, w1, w0)
    b = jnp.where(do, w0, w1)
    q00, q01 = jnp.where(do, q01, q00), jnp.where(do, q00, q01)
    q10, q11 = jnp.where(do, q11, q10), jnp.where(do, q10, q11)

    t00 = q00 * (s11 * q00 + s12 * q10) + q10 * (s12 * q00 + s22 * q10)
    t01 = q00 * (s11 * q01 + s12 * q11) + q10 * (s12 * q01 + s22 * q11)
    t10 = q01 * (s11 * q00 + s12 * q10) + q11 * (s12 * q00 + s22 * q10)
    t11 = q01 * (s11 * q01 + s12 * q11) + q11 * (s12 * q01 + s22 * q11)

    eps = 1e-8
    e00 = t00 / (2.0 * a + eps)
    e01 = t01 / (a + b + eps)
    e10 = t10 / (a + b + eps)
    e11 = t11 / (2.0 * b + eps)

    w00 = q00 * (e00 * q00 + e01 * q01) + q01 * (e10 * q00 + e11 * q01)
    w01 = q00 * (e00 * q10 + e01 * q11) + q01 * (e10 * q10 + e11 * q11)
    w10 = q10 * (e00 * q00 + e01 * q01) + q11 * (e10 * q00 + e11 * q01)
    w11 = q10 * (e00 * q10 + e01 * q11) + q11 * (e10 * q10 + e11 * q11)

    out_ref[...] = w00 * w11 - w01 * w10


def _chan(x, n):
    return x.reshape(x.shape[0], SUB, n // SUB)


def _frames_call(cov, n):
    out = pl.pallas_call(
        _frames_kernel,
        out_shape=jax.ShapeDtypeStruct((9, SUB, n // SUB), jnp.float32),
    )(_chan(cov, n))
    return out.reshape(9, n)


def _curv_call(s2, m, fr, n):
    out = pl.pallas_call(
        _curv_kernel,
        out_shape=jax.ShapeDtypeStruct((SUB, n // SUB), jnp.float32),
    )(_chan(s2, n), _chan(m, n), _chan(fr, n))
    return out.reshape(n)


def _pipeline(p):
    n = p.shape[0]
    idx = _knn_topk(p)

    knn = p[idx]
    centered = knn - knn.mean(axis=-2, keepdims=True)
    covf = jnp.matmul(jnp.swapaxes(centered, -1, -2), centered) / (K - 1)
    cov = jnp.stack([covf[:, 0, 0], covf[:, 0, 1], covf[:, 0, 2],
                     covf[:, 1, 1], covf[:, 1, 2], covf[:, 2, 2]], axis=0)

    fr = _frames_call(cov, n)
    normals = fr[0:3].T

    r = knn - p[:, None, :]
    hi = jax.lax.Precision.HIGHEST
    s2f = jnp.einsum('nkc,nkd->ncd', r, r, precision=hi)
    s2 = jnp.stack([s2f[:, 0, 0], s2f[:, 0, 1], s2f[:, 0, 2],
                    s2f[:, 1, 1], s2f[:, 1, 2], s2f[:, 2, 2]], axis=0)

    gn = normals[idx]
    nd = gn - normals[:, None, :]
    mf = jnp.einsum('nkc,nkd->ncd', nd, r, precision=hi)
    m = mf.reshape(n, 9).T

    return _curv_call(s2, m, fr, n)


def kernel(pointscloud):
    return jax.vmap(_pipeline)(pointscloud)
